# baseline (speedup 1.0000x reference)
import functools

import jax
import jax.numpy as jnp
from jax import lax
from jax.experimental import pallas as pl
from jax.experimental.pallas import tpu as pltpu
from jax.experimental.pallas import tpu_sc as plsc

VOCAB = 100000
EMBED = 128
HIDDEN = 512
OUT = 256
BATCH = 4096
HIST = 50

NC = 2
NS = 16
NW = NC * NS
ROWS_PER_W = BATCH // NW
CB = 1
CHUNK_IDX = CB * HIST
NCHUNK = ROWS_PER_W // CB
LANES = 16
EV = EMBED // LANES
NBUF = 8
DEPTH = 7

_sc_mesh = plsc.VectorSubcoreMesh(
    core_axis_name="c", subcore_axis_name="s", num_cores=NC, num_subcores=NS
)


@functools.partial(
    pl.kernel,
    out_type=jax.ShapeDtypeStruct((BATCH, EMBED), jnp.float32),
    mesh=_sc_mesh,
    scratch_types=[
        pltpu.VMEM((NCHUNK, HIST), jnp.int32),
        pltpu.VMEM((ROWS_PER_W, EMBED), jnp.float32),
    ]
    + [pltpu.VMEM((CHUNK_IDX, EMBED), jnp.float32) for _ in range(NBUF)]
    + [pltpu.SemaphoreType.DMA for _ in range(NBUF)],
)
def _embbag_sum(idx_hbm, table_hbm, out_hbm, idx_v, pool_v, *bufs_and_sems):
    bufs = tuple(zip(bufs_and_sems[:NBUF], bufs_and_sems[NBUF:]))
    wid = lax.axis_index("s") * NC + lax.axis_index("c")

    pltpu.sync_copy(idx_hbm.at[pl.ds(wid * NCHUNK, NCHUNK)], idx_v)

    def gather(c, buf, sem):
        pltpu.async_copy(table_hbm.at[idx_v.at[jnp.minimum(c, NCHUNK - 1)]],
                         buf, sem)

    def drain(buf, sem):
        pltpu.make_async_copy(table_hbm.at[idx_v.at[0]], buf, sem).wait()

    def accumulate(c, buf):
        for i in range(CB):
            def bag_body(r, accs):
                return tuple(
                    accs[j] + buf[i * HIST + r, pl.ds(j * LANES, LANES)]
                    for j in range(EV)
                )
            accs = lax.fori_loop(
                0, HIST, bag_body,
                tuple(jnp.zeros((LANES,), jnp.float32) for _ in range(EV)),
                unroll=5,
            )
            for j in range(EV):
                pool_v[c * CB + i, pl.ds(j * LANES, LANES)] = accs[j]

    for d in range(DEPTH):
        gather(d, *bufs[d])

    def ring_body(p, carry):
        a = p * NBUF
        for q in range(NBUF):
            gather(a + q + DEPTH, *bufs[(q + DEPTH) % NBUF])
            drain(*bufs[q])
            accumulate(a + q, bufs[q][0])
        return carry

    lax.fori_loop(0, NCHUNK // NBUF, ring_body, 0)
    for d in range(DEPTH):
        drain(*bufs[d])

    pltpu.sync_copy(pool_v, out_hbm.at[pl.ds(wid * ROWS_PER_W, ROWS_PER_W)])


def _mlp_body(x_ref, w1_ref, b1_ref, w2_ref, b2_ref, o_ref):
    x = x_ref[...] * (1.0 / HIST)
    h = lax.dot_general(
        x, w1_ref[...], (((1,), (1,)), ((), ())),
        preferred_element_type=jnp.float32,
    ) + b1_ref[...]
    o_ref[...] = lax.dot_general(
        h, w2_ref[...], (((1,), (1,)), ((), ())),
        preferred_element_type=jnp.float32,
    ) + b2_ref[...]


_mlp = pl.pallas_call(
    _mlp_body,
    out_shape=jax.ShapeDtypeStruct((BATCH, OUT), jnp.float32),
)


@jax.jit
def kernel(input_batch, emb_table, W1, b1, W2, b2):
    pooled_sum = _embbag_sum(input_batch.astype(jnp.int32), emb_table)
    return _mlp(pooled_sum, W1, b1.reshape(1, HIDDEN), W2, b2.reshape(1, OUT))


# speedup vs baseline: 1.0053x; 1.0053x over previous
import functools

import jax
import jax.numpy as jnp
from jax import lax
from jax.experimental import pallas as pl
from jax.experimental.pallas import tpu as pltpu
from jax.experimental.pallas import tpu_sc as plsc

VOCAB = 100000
EMBED = 128
HIDDEN = 512
OUT = 256
BATCH = 4096
HIST = 50

NC = 2
NS = 16
NW = NC * NS
ROWS_PER_W = BATCH // NW
CB = 1
CHUNK_IDX = CB * HIST
NCHUNK = ROWS_PER_W // CB
LANES = 16
EV = EMBED // LANES
NBUF = 8
DEPTH = 6

_sc_mesh = plsc.VectorSubcoreMesh(
    core_axis_name="c", subcore_axis_name="s", num_cores=NC, num_subcores=NS
)


@functools.partial(
    pl.kernel,
    out_type=jax.ShapeDtypeStruct((BATCH, EMBED), jnp.float32),
    mesh=_sc_mesh,
    scratch_types=[
        pltpu.VMEM((NCHUNK, HIST), jnp.int32),
        pltpu.VMEM((ROWS_PER_W, EMBED), jnp.float32),
    ]
    + [pltpu.VMEM((CHUNK_IDX, EMBED), jnp.float32) for _ in range(NBUF)]
    + [pltpu.SemaphoreType.DMA for _ in range(NBUF)],
)
def _embbag_sum(idx_hbm, table_hbm, out_hbm, idx_v, pool_v, *bufs_and_sems):
    bufs = tuple(zip(bufs_and_sems[:NBUF], bufs_and_sems[NBUF:]))
    wid = lax.axis_index("s") * NC + lax.axis_index("c")

    pltpu.sync_copy(idx_hbm.at[pl.ds(wid * NCHUNK, NCHUNK)], idx_v)

    def gather(c, buf, sem):
        pltpu.async_copy(table_hbm.at[idx_v.at[jnp.minimum(c, NCHUNK - 1)]],
                         buf, sem)

    def drain(buf, sem):
        pltpu.make_async_copy(table_hbm.at[idx_v.at[0]], buf, sem).wait()

    def accumulate(c, buf):
        for i in range(CB):
            def bag_body(r, accs):
                return tuple(
                    accs[j] + buf[i * HIST + r, pl.ds(j * LANES, LANES)]
                    for j in range(EV)
                )
            accs = lax.fori_loop(
                0, HIST, bag_body,
                tuple(jnp.zeros((LANES,), jnp.float32) for _ in range(EV)),
                unroll=5,
            )
            for j in range(EV):
                pool_v[c * CB + i, pl.ds(j * LANES, LANES)] = accs[j]

    for d in range(DEPTH):
        gather(d, *bufs[d])

    def ring_body(p, carry):
        a = p * NBUF
        for q in range(NBUF):
            gather(a + q + DEPTH, *bufs[(q + DEPTH) % NBUF])
            drain(*bufs[q])
            accumulate(a + q, bufs[q][0])
        return carry

    lax.fori_loop(0, NCHUNK // NBUF, ring_body, 0)
    for d in range(DEPTH):
        drain(*bufs[d])

    pltpu.sync_copy(pool_v, out_hbm.at[pl.ds(wid * ROWS_PER_W, ROWS_PER_W)])


def _mlp_body(x_ref, w1_ref, b1_ref, w2_ref, b2_ref, o_ref):
    x = x_ref[...] * (1.0 / HIST)
    h = lax.dot_general(
        x, w1_ref[...], (((1,), (1,)), ((), ())),
        preferred_element_type=jnp.float32,
    ) + b1_ref[...]
    o_ref[...] = lax.dot_general(
        h, w2_ref[...], (((1,), (1,)), ((), ())),
        preferred_element_type=jnp.float32,
    ) + b2_ref[...]


_mlp = pl.pallas_call(
    _mlp_body,
    out_shape=jax.ShapeDtypeStruct((BATCH, OUT), jnp.float32),
)


@jax.jit
def kernel(input_batch, emb_table, W1, b1, W2, b2):
    pooled_sum = _embbag_sum(input_batch.astype(jnp.int32), emb_table)
    return _mlp(pooled_sum, W1, b1.reshape(1, HIDDEN), W2, b2.reshape(1, OUT))
